# Initial kernel scaffold; baseline (speedup 1.0000x reference)
import jax
import jax.numpy as jnp
from jax.experimental import pallas as pl


def kernel(rgb_map, real_cam_idx, cal):
    raise NotImplementedError("write your pallas kernel here")


# speedup vs baseline: 1.6990x; 1.6990x over previous
import jax
import jax.numpy as jnp
from jax import lax
from jax.experimental import pallas as pl
from jax.experimental.pallas import tpu as pltpu
from jax.experimental.pallas import tpu_sc as plsc

L = 16
NC, NS = 2, 16
NW = NC * NS
BLK_RAYS = 4096
BLK_FLAT = 3 * BLK_RAYS


def _take(vec, idx):
    return jnp.take_along_axis(vec, idx, axis=0, mode="promise_in_bounds")


def _body(rgb_hbm, idx_hbm, calt_hbm, out_hbm, calt_v, idx_v, rgb_v, out_v):
    wid = lax.axis_index("s") * NC + lax.axis_index("c")
    rays_per_w = idx_hbm.shape[0] // NW
    nblocks = rays_per_w // BLK_RAYS

    pltpu.sync_copy(calt_hbm, calt_v)

    def do_block(b, carry):
        ray0 = wid * rays_per_w + b * BLK_RAYS
        pltpu.sync_copy(idx_hbm.at[pl.ds(ray0, BLK_RAYS)], idx_v)
        pltpu.sync_copy(rgb_hbm.at[pl.ds(3 * ray0, BLK_FLAT)], rgb_v)

        def do_group(u, carry2):
            lane = lax.iota(jnp.int32, L)
            cam0 = lane == 0
            three = lane * 0 + 3
            ts = [jnp.where(cam0, 1.0, calt_v[pl.ds(16 * c, L)]) for c in range(3)]
            to = [jnp.where(cam0, 0.0, calt_v[pl.ds(16 * c, L)]) for c in range(3, 6)]
            rj = [lax.div(16 * j + lane, three) for j in range(3)]
            cj = [(16 * j + lane) - 3 * rj[j] for j in range(3)]
            m0 = [c == 0 for c in cj]
            m1 = [c == 1 for c in cj]
            cam = idx_v[pl.ds(u * 16, L)]
            flat_base = u * 48
            for j in range(3):
                cam_j = _take(cam, rj[j])
                s = jnp.where(
                    m0[j],
                    _take(ts[0], cam_j),
                    jnp.where(m1[j], _take(ts[1], cam_j), _take(ts[2], cam_j)),
                )
                o = jnp.where(
                    m0[j],
                    _take(to[0], cam_j),
                    jnp.where(m1[j], _take(to[1], cam_j), _take(to[2], cam_j)),
                )
                x = rgb_v[pl.ds(flat_base + 16 * j, L)]
                out_v[pl.ds(flat_base + 16 * j, L)] = x * s + o
            return carry2

        lax.fori_loop(0, BLK_RAYS // 16, do_group, 0)
        pltpu.sync_copy(out_v, out_hbm.at[pl.ds(3 * ray0, BLK_FLAT)])
        return carry

    lax.fori_loop(0, nblocks, do_block, 0)


def kernel(rgb_map, real_cam_idx, cal):
    b = rgb_map.shape[0]
    mesh = plsc.VectorSubcoreMesh(
        core_axis_name="c", subcore_axis_name="s", num_cores=NC, num_subcores=NS
    )
    run = pl.kernel(
        _body,
        out_type=jax.ShapeDtypeStruct((3 * b,), jnp.float32),
        mesh=mesh,
        scratch_types=[
            pltpu.VMEM((96,), jnp.float32),
            pltpu.VMEM((BLK_RAYS,), jnp.int32),
            pltpu.VMEM((BLK_FLAT,), jnp.float32),
            pltpu.VMEM((BLK_FLAT,), jnp.float32),
        ],
    )
    out_flat = run(
        rgb_map.reshape(-1),
        real_cam_idx.astype(jnp.int32),
        cal.T.reshape(-1),
    )
    return out_flat.reshape(b, 3)


# speedup vs baseline: 1.7033x; 1.0026x over previous
import jax
import jax.numpy as jnp
from jax import lax
from jax.experimental import pallas as pl
from jax.experimental.pallas import tpu as pltpu
from jax.experimental.pallas import tpu_sc as plsc

L = 16
NC, NS = 2, 16
NW = NC * NS
BLK_RAYS = 4096
BLK_FLAT = 3 * BLK_RAYS


def _take(vec, idx):
    return jnp.take_along_axis(vec, idx, axis=0, mode="promise_in_bounds")


def _body(rgb_hbm, idx_hbm, calt_hbm, out_hbm, calt_v, idx_v, rgb_v, out_v):
    wid = lax.axis_index("s") * NC + lax.axis_index("c")
    rays_per_w = idx_hbm.shape[0] // NW
    nblocks = rays_per_w // BLK_RAYS

    pltpu.sync_copy(calt_hbm, calt_v)
    lane = lax.iota(jnp.int32, L)
    cam0 = lane == 0
    three = lane * 0 + 3
    ts = [jnp.where(cam0, 1.0, calt_v[pl.ds(16 * c, L)]) for c in range(3)]
    to = [jnp.where(cam0, 0.0, calt_v[pl.ds(16 * c, L)]) for c in range(3, 6)]
    rj = [lax.div(16 * j + lane, three) for j in range(3)]
    cj = [(16 * j + lane) - 3 * rj[j] for j in range(3)]
    m0 = [c == 0 for c in cj]
    m1 = [c == 1 for c in cj]

    def do_block(b, carry):
        ray0 = wid * rays_per_w + b * BLK_RAYS
        pltpu.sync_copy(idx_hbm.at[pl.ds(ray0, BLK_RAYS)], idx_v)
        pltpu.sync_copy(rgb_hbm.at[pl.ds(3 * ray0, BLK_FLAT)], rgb_v)

        def do_group(u, carry2):
            cam = idx_v[pl.ds(u * 16, L)]
            flat_base = u * 48
            for j in range(3):
                cam_j = _take(cam, rj[j])
                s = jnp.where(
                    m0[j],
                    _take(ts[0], cam_j),
                    jnp.where(m1[j], _take(ts[1], cam_j), _take(ts[2], cam_j)),
                )
                o = jnp.where(
                    m0[j],
                    _take(to[0], cam_j),
                    jnp.where(m1[j], _take(to[1], cam_j), _take(to[2], cam_j)),
                )
                x = rgb_v[pl.ds(flat_base + 16 * j, L)]
                out_v[pl.ds(flat_base + 16 * j, L)] = x * s + o
            return carry2

        lax.fori_loop(0, BLK_RAYS // 16, do_group, 0)
        pltpu.sync_copy(out_v, out_hbm.at[pl.ds(3 * ray0, BLK_FLAT)])
        return carry

    lax.fori_loop(0, nblocks, do_block, 0)


def kernel(rgb_map, real_cam_idx, cal):
    b = rgb_map.shape[0]
    mesh = plsc.VectorSubcoreMesh(
        core_axis_name="c", subcore_axis_name="s", num_cores=NC, num_subcores=NS
    )
    run = pl.kernel(
        _body,
        out_type=jax.ShapeDtypeStruct((3 * b,), jnp.float32),
        mesh=mesh,
        scratch_types=[
            pltpu.VMEM((96,), jnp.float32),
            pltpu.VMEM((BLK_RAYS,), jnp.int32),
            pltpu.VMEM((BLK_FLAT,), jnp.float32),
            pltpu.VMEM((BLK_FLAT,), jnp.float32),
        ],
    )
    out_flat = run(
        rgb_map.reshape(-1),
        real_cam_idx.astype(jnp.int32),
        cal.T.reshape(-1),
    )
    return out_flat.reshape(b, 3)
